# baseline (speedup 1.0000x reference)
import jax
import jax.numpy as jnp
from jax.experimental import pallas as pl
from jax.experimental.pallas import tpu as pltpu

IN_F = 128
HID_F = 128
OUT_F = 128
N_NGBRS = 320000

K_STREAMS = 4
BLOCK = 8000
NUM_STEPS = N_NGBRS // (K_STREAMS * BLOCK)

_SLOPE = 0.01


def _lrelu(x):
    return jnp.maximum(x, _SLOPE * x)


def _dot(a, b):
    return jax.lax.dot_general(
        a, b, (((1,), (0,)), ((), ())), preferred_element_type=jnp.float32)


def _pinsage_kernel(*refs):
    x_refs = refs[:K_STREAMS]
    a_refs = refs[K_STREAMS:2 * K_STREAMS]
    qt_ref, qb_ref, hn_ref, wt_ref, wb_ref, out_ref, acc_ref, asum_ref = \
        refs[2 * K_STREAMS:]
    i = pl.program_id(0)

    @pl.when(i == 0)
    def _():
        acc_ref[...] = jnp.zeros_like(acc_ref)
        asum_ref[0, 0] = 0.0

    qt = qt_ref[...].astype(jnp.bfloat16)
    qb = qb_ref[...]
    acc = acc_ref[...]
    asum = asum_ref[0, 0]
    for k in range(K_STREAMS):
        a = a_refs[k][...].reshape(1, BLOCK)
        x16 = x_refs[k][...].astype(jnp.bfloat16)
        z = _dot(x16, qt) + qb
        l16 = _lrelu(z).astype(jnp.bfloat16)
        a16 = a.astype(jnp.bfloat16)
        acc = acc + _dot(a16, l16)
        asum = asum + jnp.sum(a)
    acc_ref[...] = acc
    asum_ref[0, 0] = asum

    @pl.when(i == NUM_STEPS - 1)
    def _():
        s = asum_ref[0, 0]
        ssafe = jnp.where(s == 0.0, 1.0, s)
        h_agg = acc_ref[...] / ssafe

        wt = wt_ref[...]
        z2 = _dot(hn_ref[...], wt[:IN_F, :]) + _dot(h_agg, wt[IN_F:, :]) \
            + wb_ref[...]
        h_two = _lrelu(z2)
        nrm = jnp.sqrt(jnp.sum(h_two * h_two))
        nsafe = jnp.where(nrm == 0.0, 1.0, nrm)
        out_ref[...] = h_two / nsafe


@jax.jit
def kernel(h_node, h_ngbrs, alpha, Q_w, Q_b, W_w, W_b):
    alpha_rows = alpha.reshape(K_STREAMS, NUM_STEPS, 1, BLOCK)

    x_specs = [
        pl.BlockSpec((BLOCK, IN_F), lambda i, k=k: (k * NUM_STEPS + i, 0))
        for k in range(K_STREAMS)
    ]
    a_specs = [
        pl.BlockSpec((1, 1, 1, BLOCK), lambda i, k=k: (k, i, 0, 0))
        for k in range(K_STREAMS)
    ]
    out = pl.pallas_call(
        _pinsage_kernel,
        grid=(NUM_STEPS,),
        in_specs=x_specs + a_specs + [
            pl.BlockSpec((IN_F, HID_F), lambda i: (0, 0)),
            pl.BlockSpec((1, HID_F), lambda i: (0, 0)),
            pl.BlockSpec((1, IN_F), lambda i: (0, 0)),
            pl.BlockSpec((IN_F + HID_F, OUT_F), lambda i: (0, 0)),
            pl.BlockSpec((1, OUT_F), lambda i: (0, 0)),
        ],
        out_specs=pl.BlockSpec((1, OUT_F), lambda i: (0, 0)),
        out_shape=jax.ShapeDtypeStruct((1, OUT_F), jnp.float32),
        scratch_shapes=[
            pltpu.VMEM((1, HID_F), jnp.float32),
            pltpu.SMEM((1, 1), jnp.float32),
        ],
    )(
        *([h_ngbrs] * K_STREAMS),
        *([alpha_rows] * K_STREAMS),
        Q_w.T,
        Q_b.reshape(1, HID_F),
        h_node.reshape(1, IN_F),
        W_w.T,
        W_b.reshape(1, OUT_F),
    )
    return out[0]


# speedup vs baseline: 1.1296x; 1.1296x over previous
import jax
import jax.numpy as jnp
from jax.experimental import pallas as pl
from jax.experimental.pallas import tpu as pltpu

IN_F = 128
HID_F = 128
OUT_F = 128
N_NGBRS = 320000

K_STREAMS = 4
BLOCK = 16000
NUM_STEPS = N_NGBRS // (K_STREAMS * BLOCK)

_SLOPE = 0.01


def _lrelu(x):
    return jnp.maximum(x, _SLOPE * x)


def _dot(a, b):
    return jax.lax.dot_general(
        a, b, (((1,), (0,)), ((), ())), preferred_element_type=jnp.float32)


def _pinsage_kernel(*refs):
    x_refs = refs[:K_STREAMS]
    a_refs = refs[K_STREAMS:2 * K_STREAMS]
    qt_ref, qb_ref, hn_ref, wt_ref, wb_ref, out_ref, acc_ref, asum_ref = \
        refs[2 * K_STREAMS:]
    i = pl.program_id(0)

    @pl.when(i == 0)
    def _():
        acc_ref[...] = jnp.zeros_like(acc_ref)
        asum_ref[0, 0] = 0.0

    qt = qt_ref[...]
    qb = qb_ref[...]
    acc = acc_ref[...]
    asum = asum_ref[0, 0]
    for k in range(K_STREAMS):
        a = a_refs[k][...].reshape(1, BLOCK)
        z = _dot(x_refs[k][...], qt) + qb
        l16 = _lrelu(z).astype(jnp.bfloat16)
        a16 = a.astype(jnp.bfloat16)
        acc = acc + _dot(a16, l16)
        asum = asum + jnp.sum(a)
    acc_ref[...] = acc
    asum_ref[0, 0] = asum

    @pl.when(i == NUM_STEPS - 1)
    def _():
        s = asum_ref[0, 0]
        ssafe = jnp.where(s == 0.0, 1.0, s)
        h_agg = acc_ref[...] / ssafe

        wt = wt_ref[...]
        z2 = _dot(hn_ref[...], wt[:IN_F, :]) + _dot(h_agg, wt[IN_F:, :]) \
            + wb_ref[...]
        h_two = _lrelu(z2)
        nrm = jnp.sqrt(jnp.sum(h_two * h_two))
        nsafe = jnp.where(nrm == 0.0, 1.0, nrm)
        out_ref[...] = h_two / nsafe


@jax.jit
def kernel(h_node, h_ngbrs, alpha, Q_w, Q_b, W_w, W_b):
    alpha_rows = alpha.reshape(K_STREAMS, NUM_STEPS, 1, BLOCK)

    x_specs = [
        pl.BlockSpec((BLOCK, IN_F), lambda i, k=k: (k * NUM_STEPS + i, 0))
        for k in range(K_STREAMS)
    ]
    a_specs = [
        pl.BlockSpec((1, 1, 1, BLOCK), lambda i, k=k: (k, i, 0, 0))
        for k in range(K_STREAMS)
    ]
    out = pl.pallas_call(
        _pinsage_kernel,
        grid=(NUM_STEPS,),
        in_specs=x_specs + a_specs + [
            pl.BlockSpec((IN_F, HID_F), lambda i: (0, 0)),
            pl.BlockSpec((1, HID_F), lambda i: (0, 0)),
            pl.BlockSpec((1, IN_F), lambda i: (0, 0)),
            pl.BlockSpec((IN_F + HID_F, OUT_F), lambda i: (0, 0)),
            pl.BlockSpec((1, OUT_F), lambda i: (0, 0)),
        ],
        out_specs=pl.BlockSpec((1, OUT_F), lambda i: (0, 0)),
        out_shape=jax.ShapeDtypeStruct((1, OUT_F), jnp.float32),
        compiler_params=pltpu.CompilerParams(
            vmem_limit_bytes=128 * 1024 * 1024),
        scratch_shapes=[
            pltpu.VMEM((1, HID_F), jnp.float32),
            pltpu.SMEM((1, 1), jnp.float32),
        ],
    )(
        *([h_ngbrs] * K_STREAMS),
        *([alpha_rows] * K_STREAMS),
        Q_w.T,
        Q_b.reshape(1, HID_F),
        h_node.reshape(1, IN_F),
        W_w.T,
        W_b.reshape(1, OUT_F),
    )
    return out[0]


# speedup vs baseline: 1.1570x; 1.0243x over previous
import jax
import jax.numpy as jnp
from jax.experimental import pallas as pl
from jax.experimental.pallas import tpu as pltpu

IN_F = 128
HID_F = 128
OUT_F = 128
N_NGBRS = 320000

K_STREAMS = 4
BLOCK = 16000
NUM_STEPS = N_NGBRS // (K_STREAMS * BLOCK)

_SLOPE = 0.01


def _lrelu(x):
    return jnp.maximum(x, _SLOPE * x)


def _dot(a, b):
    return jax.lax.dot_general(
        a, b, (((1,), (0,)), ((), ())), preferred_element_type=jnp.float32)


def _pinsage_kernel(*refs):
    x_refs = refs[:K_STREAMS]
    a_refs = refs[K_STREAMS:2 * K_STREAMS]
    qt_ref, qb_ref, hn_ref, wt_ref, wb_ref, out_ref, acc_ref, asum_ref = \
        refs[2 * K_STREAMS:]
    i = pl.program_id(0)

    @pl.when(i == 0)
    def _():
        acc_ref[...] = jnp.zeros_like(acc_ref)
        asum_ref[0, 0] = 0.0

    qt = qt_ref[...].astype(jnp.bfloat16)
    qb = qb_ref[...]
    acc = acc_ref[...]
    asum = asum_ref[0, 0]
    for k in range(K_STREAMS):
        a = a_refs[k][...].reshape(1, BLOCK)
        z = _dot(x_refs[k][...].astype(jnp.bfloat16), qt) + qb
        l16 = _lrelu(z).astype(jnp.bfloat16)
        a16 = a.astype(jnp.bfloat16)
        acc = acc + _dot(a16, l16)
        asum = asum + jnp.sum(a)
    acc_ref[...] = acc
    asum_ref[0, 0] = asum

    @pl.when(i == NUM_STEPS - 1)
    def _():
        s = asum_ref[0, 0]
        ssafe = jnp.where(s == 0.0, 1.0, s)
        h_agg = acc_ref[...] / ssafe

        wt = wt_ref[...]
        z2 = _dot(hn_ref[...], wt[:IN_F, :]) + _dot(h_agg, wt[IN_F:, :]) \
            + wb_ref[...]
        h_two = _lrelu(z2)
        nrm = jnp.sqrt(jnp.sum(h_two * h_two))
        nsafe = jnp.where(nrm == 0.0, 1.0, nrm)
        out_ref[...] = h_two / nsafe


@jax.jit
def kernel(h_node, h_ngbrs, alpha, Q_w, Q_b, W_w, W_b):
    alpha_rows = alpha.reshape(K_STREAMS, NUM_STEPS, 1, BLOCK)

    x_specs = [
        pl.BlockSpec((BLOCK, IN_F), lambda i, k=k: (k * NUM_STEPS + i, 0))
        for k in range(K_STREAMS)
    ]
    a_specs = [
        pl.BlockSpec((1, 1, 1, BLOCK), lambda i, k=k: (k, i, 0, 0))
        for k in range(K_STREAMS)
    ]
    out = pl.pallas_call(
        _pinsage_kernel,
        grid=(NUM_STEPS,),
        in_specs=x_specs + a_specs + [
            pl.BlockSpec((IN_F, HID_F), lambda i: (0, 0)),
            pl.BlockSpec((1, HID_F), lambda i: (0, 0)),
            pl.BlockSpec((1, IN_F), lambda i: (0, 0)),
            pl.BlockSpec((IN_F + HID_F, OUT_F), lambda i: (0, 0)),
            pl.BlockSpec((1, OUT_F), lambda i: (0, 0)),
        ],
        out_specs=pl.BlockSpec((1, OUT_F), lambda i: (0, 0)),
        out_shape=jax.ShapeDtypeStruct((1, OUT_F), jnp.float32),
        compiler_params=pltpu.CompilerParams(
            vmem_limit_bytes=128 * 1024 * 1024),
        scratch_shapes=[
            pltpu.VMEM((1, HID_F), jnp.float32),
            pltpu.SMEM((1, 1), jnp.float32),
        ],
    )(
        *([h_ngbrs] * K_STREAMS),
        *([alpha_rows] * K_STREAMS),
        Q_w.T,
        Q_b.reshape(1, HID_F),
        h_node.reshape(1, IN_F),
        W_w.T,
        W_b.reshape(1, OUT_F),
    )
    return out[0]


# speedup vs baseline: 1.3108x; 1.1329x over previous
import jax
import jax.numpy as jnp
from jax.experimental import pallas as pl
from jax.experimental.pallas import tpu as pltpu

IN_F = 128
HID_F = 128
OUT_F = 128
N_NGBRS = 320000

CHMAX = 16000
NBUF = 6
SIZES = [2048, 2048, 4096, 8192] + [16000] * 18 + [15616]
OFFS = []
_o = 0
for _s in SIZES:
    OFFS.append(_o)
    _o += _s
assert _o == N_NGBRS
NCHUNK = len(SIZES)

_SLOPE = 0.01


def _lrelu(x):
    return jnp.maximum(x, _SLOPE * x)


def _dot(a, b):
    return jax.lax.dot_general(
        a, b, (((1,), (0,)), ((), ())), preferred_element_type=jnp.float32)


def _pinsage_kernel(x_hbm, a_hbm, qt_ref, qb_ref, hn_ref, wt_ref, wb_ref,
                    out_ref, xbuf, abuf, xsem, asem):
    def _x_copy(c):
        b = c % NBUF
        return pltpu.make_async_copy(
            x_hbm.at[pl.ds(OFFS[c], SIZES[c]), :],
            xbuf.at[b, pl.ds(0, SIZES[c]), :],
            xsem.at[b])

    def _a_copy(c):
        b = c % NBUF
        return pltpu.make_async_copy(
            a_hbm.at[:, pl.ds(OFFS[c], SIZES[c])],
            abuf.at[b, :, pl.ds(0, SIZES[c])],
            asem.at[b])

    for c in range(NBUF):
        _x_copy(c).start()
        _a_copy(c).start()

    qt = qt_ref[...].astype(jnp.bfloat16)
    qb = qb_ref[...]
    acc = jnp.zeros((1, HID_F), jnp.float32)
    asum = jnp.float32(0.0)
    for c in range(NCHUNK):
        b = c % NBUF
        _x_copy(c).wait()
        _a_copy(c).wait()
        x = xbuf[b, 0:SIZES[c], :]
        a = abuf[b, :, 0:SIZES[c]]
        z = _dot(x.astype(jnp.bfloat16), qt) + qb
        l16 = _lrelu(z).astype(jnp.bfloat16)
        acc = acc + _dot(a.astype(jnp.bfloat16), l16)
        asum = asum + jnp.sum(a)
        if c + NBUF < NCHUNK:
            _x_copy(c + NBUF).start()
            _a_copy(c + NBUF).start()

    ssafe = jnp.where(asum == 0.0, 1.0, asum)
    h_agg = acc / ssafe
    wt = wt_ref[...]
    z2 = _dot(hn_ref[...], wt[:IN_F, :]) + _dot(h_agg, wt[IN_F:, :]) \
        + wb_ref[...]
    h_two = _lrelu(z2)
    nrm = jnp.sqrt(jnp.sum(h_two * h_two))
    nsafe = jnp.where(nrm == 0.0, 1.0, nrm)
    out_ref[...] = h_two / nsafe


@jax.jit
def kernel(h_node, h_ngbrs, alpha, Q_w, Q_b, W_w, W_b):
    out = pl.pallas_call(
        _pinsage_kernel,
        in_specs=[
            pl.BlockSpec(memory_space=pltpu.MemorySpace.HBM),
            pl.BlockSpec(memory_space=pltpu.MemorySpace.HBM),
            pl.BlockSpec(memory_space=pltpu.MemorySpace.VMEM),
            pl.BlockSpec(memory_space=pltpu.MemorySpace.VMEM),
            pl.BlockSpec(memory_space=pltpu.MemorySpace.VMEM),
            pl.BlockSpec(memory_space=pltpu.MemorySpace.VMEM),
            pl.BlockSpec(memory_space=pltpu.MemorySpace.VMEM),
        ],
        out_specs=pl.BlockSpec(memory_space=pltpu.MemorySpace.VMEM),
        out_shape=jax.ShapeDtypeStruct((1, OUT_F), jnp.float32),
        compiler_params=pltpu.CompilerParams(
            vmem_limit_bytes=128 * 1024 * 1024),
        scratch_shapes=[
            pltpu.VMEM((NBUF, CHMAX, IN_F), jnp.float32),
            pltpu.VMEM((NBUF, 1, CHMAX), jnp.float32),
            pltpu.SemaphoreType.DMA((NBUF,)),
            pltpu.SemaphoreType.DMA((NBUF,)),
        ],
    )(
        h_ngbrs,
        alpha.reshape(1, N_NGBRS),
        Q_w.T,
        Q_b.reshape(1, HID_F),
        h_node.reshape(1, IN_F),
        W_w.T,
        W_b.reshape(1, OUT_F),
    )
    return out[0]


# speedup vs baseline: 1.3179x; 1.0053x over previous
import jax
import jax.numpy as jnp
from jax.experimental import pallas as pl
from jax.experimental.pallas import tpu as pltpu

IN_F = 128
HID_F = 128
OUT_F = 128
N_NGBRS = 320000

CHMAX = 16000
NBUF = 7
SIZES = [2048, 2048, 4096, 8192] + [16000] * 18 + [15616]
OFFS = []
_o = 0
for _s in SIZES:
    OFFS.append(_o)
    _o += _s
assert _o == N_NGBRS
NCHUNK = len(SIZES)

_SLOPE = 0.01


def _lrelu(x):
    return jnp.maximum(x, _SLOPE * x)


def _dot(a, b):
    return jax.lax.dot_general(
        a, b, (((1,), (0,)), ((), ())), preferred_element_type=jnp.float32)


def _pinsage_kernel(x_hbm, a_hbm, qt_ref, qb_ref, hn_ref, wt_ref, wb_ref,
                    out_ref, xbuf, abuf, xsem, asem):
    def _x_copy(c):
        b = c % NBUF
        return pltpu.make_async_copy(
            x_hbm.at[pl.ds(OFFS[c], SIZES[c]), :],
            xbuf.at[b, pl.ds(0, SIZES[c]), :],
            xsem.at[b])

    def _a_copy(c):
        b = c % NBUF
        return pltpu.make_async_copy(
            a_hbm.at[:, pl.ds(OFFS[c], SIZES[c])],
            abuf.at[b, :, pl.ds(0, SIZES[c])],
            asem.at[b])

    for c in range(NBUF):
        _x_copy(c).start()
        _a_copy(c).start()

    qt = qt_ref[...].astype(jnp.bfloat16)
    qb = qb_ref[...]
    acc = jnp.zeros((1, HID_F), jnp.float32)
    asum = jnp.float32(0.0)
    for c in range(NCHUNK):
        b = c % NBUF
        _x_copy(c).wait()
        _a_copy(c).wait()
        x = xbuf[b, 0:SIZES[c], :]
        a = abuf[b, :, 0:SIZES[c]]
        z = _dot(x.astype(jnp.bfloat16), qt) + qb
        l16 = _lrelu(z).astype(jnp.bfloat16)
        acc = acc + _dot(a.astype(jnp.bfloat16), l16)
        asum = asum + jnp.sum(a)
        if c + NBUF < NCHUNK:
            _x_copy(c + NBUF).start()
            _a_copy(c + NBUF).start()

    ssafe = jnp.where(asum == 0.0, 1.0, asum)
    h_agg = acc / ssafe
    wt = wt_ref[...]
    z2 = _dot(hn_ref[...], wt[:IN_F, :]) + _dot(h_agg, wt[IN_F:, :]) \
        + wb_ref[...]
    h_two = _lrelu(z2)
    nrm = jnp.sqrt(jnp.sum(h_two * h_two))
    nsafe = jnp.where(nrm == 0.0, 1.0, nrm)
    out_ref[...] = h_two / nsafe


@jax.jit
def kernel(h_node, h_ngbrs, alpha, Q_w, Q_b, W_w, W_b):
    out = pl.pallas_call(
        _pinsage_kernel,
        in_specs=[
            pl.BlockSpec(memory_space=pltpu.MemorySpace.HBM),
            pl.BlockSpec(memory_space=pltpu.MemorySpace.HBM),
            pl.BlockSpec(memory_space=pltpu.MemorySpace.VMEM),
            pl.BlockSpec(memory_space=pltpu.MemorySpace.VMEM),
            pl.BlockSpec(memory_space=pltpu.MemorySpace.VMEM),
            pl.BlockSpec(memory_space=pltpu.MemorySpace.VMEM),
            pl.BlockSpec(memory_space=pltpu.MemorySpace.VMEM),
        ],
        out_specs=pl.BlockSpec(memory_space=pltpu.MemorySpace.VMEM),
        out_shape=jax.ShapeDtypeStruct((1, OUT_F), jnp.float32),
        compiler_params=pltpu.CompilerParams(
            vmem_limit_bytes=128 * 1024 * 1024),
        scratch_shapes=[
            pltpu.VMEM((NBUF, CHMAX, IN_F), jnp.float32),
            pltpu.VMEM((NBUF, 1, CHMAX), jnp.float32),
            pltpu.SemaphoreType.DMA((NBUF,)),
            pltpu.SemaphoreType.DMA((NBUF,)),
        ],
    )(
        h_ngbrs,
        alpha.reshape(1, N_NGBRS),
        Q_w.T,
        Q_b.reshape(1, HID_F),
        h_node.reshape(1, IN_F),
        W_w.T,
        W_b.reshape(1, OUT_F),
    )
    return out[0]


# speedup vs baseline: 1.3179x; 1.0000x over previous
import jax
import jax.numpy as jnp
from jax.experimental import pallas as pl
from jax.experimental.pallas import tpu as pltpu

IN_F = 128
HID_F = 128
OUT_F = 128
N_NGBRS = 320000

CHMAX = 16000
NBUF = 7
SIZES = [2048, 2048, 4096, 8192] + [16000] * 18 + [15616]
OFFS = []
_o = 0
for _s in SIZES:
    OFFS.append(_o)
    _o += _s
assert _o == N_NGBRS
NCHUNK = len(SIZES)

_SLOPE = 0.01


def _lrelu(x):
    return jnp.maximum(x, _SLOPE * x)


def _dot(a, b):
    return jax.lax.dot_general(
        a, b, (((1,), (0,)), ((), ())), preferred_element_type=jnp.float32)


def _pinsage_kernel(x_hbm, a_hbm, qt_ref, qb_ref, hn_ref, wt_ref, wb_ref,
                    out_ref, xbuf, abuf, xsem, asem):
    def _x_copy(c):
        b = c % NBUF
        return pltpu.make_async_copy(
            x_hbm.at[pl.ds(OFFS[c], SIZES[c]), :],
            xbuf.at[b, pl.ds(0, SIZES[c]), :],
            xsem.at[b])

    def _a_copy(c):
        b = c % NBUF
        return pltpu.make_async_copy(
            a_hbm.at[:, pl.ds(OFFS[c], SIZES[c])],
            abuf.at[b, :, pl.ds(0, SIZES[c])],
            asem.at[b])

    for c in range(NBUF):
        _x_copy(c).start()
        _a_copy(c).start()

    qt = qt_ref[...].astype(jnp.bfloat16)
    qb16 = qb_ref[...].astype(jnp.bfloat16)
    acc = jnp.zeros((1, HID_F), jnp.float32)
    asum = jnp.float32(0.0)
    for c in range(NCHUNK):
        b = c % NBUF
        _x_copy(c).wait()
        _a_copy(c).wait()
        x = xbuf[b, 0:SIZES[c], :]
        a = abuf[b, :, 0:SIZES[c]]
        z16 = _dot(x.astype(jnp.bfloat16), qt).astype(jnp.bfloat16) + qb16
        l16 = _lrelu(z16)
        acc = acc + _dot(a.astype(jnp.bfloat16), l16)
        asum = asum + jnp.sum(a)
        if c + NBUF < NCHUNK:
            _x_copy(c + NBUF).start()
            _a_copy(c + NBUF).start()

    ssafe = jnp.where(asum == 0.0, 1.0, asum)
    h_agg = acc / ssafe
    wt = wt_ref[...]
    z2 = _dot(hn_ref[...], wt[:IN_F, :]) + _dot(h_agg, wt[IN_F:, :]) \
        + wb_ref[...]
    h_two = _lrelu(z2)
    nrm = jnp.sqrt(jnp.sum(h_two * h_two))
    nsafe = jnp.where(nrm == 0.0, 1.0, nrm)
    out_ref[...] = h_two / nsafe


@jax.jit
def kernel(h_node, h_ngbrs, alpha, Q_w, Q_b, W_w, W_b):
    out = pl.pallas_call(
        _pinsage_kernel,
        in_specs=[
            pl.BlockSpec(memory_space=pltpu.MemorySpace.HBM),
            pl.BlockSpec(memory_space=pltpu.MemorySpace.HBM),
            pl.BlockSpec(memory_space=pltpu.MemorySpace.VMEM),
            pl.BlockSpec(memory_space=pltpu.MemorySpace.VMEM),
            pl.BlockSpec(memory_space=pltpu.MemorySpace.VMEM),
            pl.BlockSpec(memory_space=pltpu.MemorySpace.VMEM),
            pl.BlockSpec(memory_space=pltpu.MemorySpace.VMEM),
        ],
        out_specs=pl.BlockSpec(memory_space=pltpu.MemorySpace.VMEM),
        out_shape=jax.ShapeDtypeStruct((1, OUT_F), jnp.float32),
        compiler_params=pltpu.CompilerParams(
            vmem_limit_bytes=128 * 1024 * 1024),
        scratch_shapes=[
            pltpu.VMEM((NBUF, CHMAX, IN_F), jnp.float32),
            pltpu.VMEM((NBUF, 1, CHMAX), jnp.float32),
            pltpu.SemaphoreType.DMA((NBUF,)),
            pltpu.SemaphoreType.DMA((NBUF,)),
        ],
    )(
        h_ngbrs,
        alpha.reshape(1, N_NGBRS),
        Q_w.T,
        Q_b.reshape(1, HID_F),
        h_node.reshape(1, IN_F),
        W_w.T,
        W_b.reshape(1, OUT_F),
    )
    return out[0]
